# baseline (speedup 1.0000x reference)
import functools

import jax
import jax.numpy as jnp
from jax import lax
from jax.experimental import pallas as pl
from jax.experimental.pallas import tpu as pltpu

VOCAB = 100000
D = 64
K_MEM = 120000
SHORTLIST_K = 64
T_TRAIN = 4
BAND = 4
BETA_START, BETA_END = 1.0, 5.0
TAU_START, TAU_END = 1.0, 0.1
EARLY_EXIT_TOL = 1e-4
B, S = 32, 8
NTOK = B * S

SCORE_TILE = 512
LOGIT_TILE = 512


def _matmul_nt_kernel(a_ref, b_ref, o_ref):
    o_ref[...] = lax.dot_general(
        a_ref[...], b_ref[...],
        dimension_numbers=(((1,), (1,)), ((), ())),
        preferred_element_type=jnp.float32,
    )


def _matmul_nt(a, b_table, n_cols, tile):
    m = a.shape[0]
    grid = pl.cdiv(n_cols, tile)
    return pl.pallas_call(
        _matmul_nt_kernel,
        grid=(grid,),
        in_specs=[
            pl.BlockSpec((m, D), lambda i: (0, 0)),
            pl.BlockSpec((tile, D), lambda i: (i, 0)),
        ],
        out_specs=pl.BlockSpec((m, tile), lambda i: (0, i)),
        out_shape=jax.ShapeDtypeStruct((m, n_cols), jnp.float32),
    )(a, b_table)


def kernel(tokens, embed_table, memory, field_w, scout_w):
    E = jnp.take(embed_table, tokens.reshape(-1), axis=0)

    scores = _matmul_nt(E, memory, K_MEM, SCORE_TILE)
    _, Kset = lax.top_k(scores.reshape(B, S, K_MEM), SHORTLIST_K)
    cand = jnp.take(memory, Kset, axis=0)

    Eb = E.reshape(B, S, D)
    proj = jnp.einsum('bsd,de->bse', Eb, scout_w)
    sims = jnp.einsum('bse,bske->bsk', proj, cand)
    P = jax.nn.softmax(sims, axis=-1)
    Y = jnp.einsum('bsk,bske->bse', P, cand)
    Lam = jnp.zeros_like(Y)
    W = field_w / jnp.maximum(1.0, jnp.linalg.norm(field_w))
    X = Eb

    def field_apply(Yv):
        out = jnp.zeros_like(Yv)
        for i, off in enumerate(range(-BAND, BAND + 1)):
            out = out + jnp.roll(Yv, shift=off, axis=1) * W[i]
        return out

    prev_energy = jnp.array(jnp.inf, dtype=jnp.float32)
    done = jnp.array(False)
    for t in range(T_TRAIN):
        frac = t / max(T_TRAIN - 1, 1)
        beta = BETA_START + frac * (BETA_END - BETA_START)
        tau = TAU_START + frac * (TAU_END - TAU_START)
        FY = field_apply(Y)
        Lam_n = Lam + tau * (FY - X)
        resid = Y - X + Lam_n
        g = jnp.einsum('bse,bske->bsk', resid, cand)
        P_n = jax.nn.softmax(jnp.log(P + 1e-9) - beta * g, axis=-1)
        Y_n = jnp.einsum('bsk,bske->bse', P_n, cand)
        energy = 0.5 * jnp.mean(jnp.sum((Y_n - X) ** 2, axis=-1)) \
               + 0.5 * jnp.mean(jnp.sum((field_apply(Y_n) - X) ** 2, axis=-1))
        e = energy.astype(jnp.float32)
        Lam = jnp.where(done, Lam, Lam_n)
        P = jnp.where(done, P, P_n)
        Y = jnp.where(done, Y, Y_n)
        if t > 0:
            rel = jnp.abs(prev_energy - e) / jnp.maximum(jnp.abs(prev_energy), 1e-6)
            done = done | (rel <= EARLY_EXIT_TOL)
        prev_energy = jnp.where(done, prev_energy, e)

    Yf = Y.reshape(NTOK, D)
    logits = _matmul_nt(Yf, memory, VOCAB, LOGIT_TILE)
    return logits.reshape(B, S, VOCAB)


# speedup vs baseline: 22.4069x; 22.4069x over previous
import functools

import jax
import jax.numpy as jnp
from jax import lax
from jax.experimental import pallas as pl
from jax.experimental.pallas import tpu as pltpu

VOCAB = 100000
D = 64
K_MEM = 120000
SHORTLIST_K = 64
T_TRAIN = 4
BAND = 4
BETA_START, BETA_END = 1.0, 5.0
TAU_START, TAU_END = 1.0, 0.1
EARLY_EXIT_TOL = 1e-4
B, S = 32, 8
NTOK = B * S

SCORE_TILE = 512
BLK = 64
N_TILES = 235
K_PAD = N_TILES * SCORE_TILE
NB = K_PAD // BLK
NB_OUT = 1888
NEG_INF = float("-inf")

LOGIT_TILE = 512


def _skey(x):
    b = lax.bitcast_convert_type(x, jnp.int32)
    return jnp.where(b >= 0, b, b ^ jnp.int32(0x7FFFFFFF))


def _score_bmax_kernel(e_ref, mem_ref, s_ref, bm_ref):
    i = pl.program_id(0)
    s = lax.dot_general(
        e_ref[...], mem_ref[...],
        dimension_numbers=(((1,), (1,)), ((), ())),
        preferred_element_type=jnp.float32,
    )
    col = i * SCORE_TILE + lax.broadcasted_iota(jnp.int32, s.shape, 1)
    s = jnp.where(col < K_MEM, s, NEG_INF)
    s_ref[...] = s
    nblk = SCORE_TILE // BLK
    parts = [jnp.max(s[:, j * BLK:(j + 1) * BLK], axis=1, keepdims=True)
             for j in range(nblk)]
    bm = jnp.concatenate(parts, axis=1)
    off = pl.multiple_of(i * nblk, 8)
    bm_ref[pl.ds(off, nblk), :] = bm.T

    @pl.when(i == 0)
    def _fill_tail():
        bm_ref[pl.ds(NB, NB_OUT - NB), :] = jnp.full(
            (NB_OUT - NB, NTOK), NEG_INF, jnp.float32)


def _scores_and_blockmax(E, memory):
    return pl.pallas_call(
        _score_bmax_kernel,
        grid=(N_TILES,),
        in_specs=[
            pl.BlockSpec((NTOK, D), lambda i: (0, 0)),
            pl.BlockSpec((SCORE_TILE, D), lambda i: (i, 0)),
        ],
        out_specs=[
            pl.BlockSpec((NTOK, SCORE_TILE), lambda i: (0, i)),
            pl.BlockSpec((NB_OUT, NTOK), lambda i: (0, 0)),
        ],
        out_shape=[
            jax.ShapeDtypeStruct((NTOK, K_PAD), jnp.float32),
            jax.ShapeDtypeStruct((NB_OUT, NTOK), jnp.float32),
        ],
    )(E, memory)


def _bisect_kernel(x_ref, o_ref, *, kwant, axis):
    keys = _skey(x_ref[...])
    oshape = o_ref.shape

    def body(_, lohi):
        lo, hi = lohi
        mid = (lo >> 1) + (hi >> 1) + (lo & hi & 1)
        cnt = jnp.sum((keys >= mid).astype(jnp.int32), axis=axis, keepdims=True)
        ge = cnt >= kwant
        return jnp.where(ge, mid, lo), jnp.where(ge, hi, mid)

    lo0 = jnp.full(oshape, jnp.int32(-2147483648))
    hi0 = jnp.full(oshape, jnp.int32(2147483647))
    lo, _ = lax.fori_loop(0, 32, body, (lo0, hi0))
    o_ref[...] = lo


def _kth_largest_key(x, kwant, axis):
    m, n = x.shape
    oshape = (m, 1) if axis == 1 else (1, n)
    return pl.pallas_call(
        functools.partial(_bisect_kernel, kwant=kwant, axis=axis),
        out_shape=jax.ShapeDtypeStruct(oshape, jnp.int32),
    )(x)


def _compact64(mask):
    cs = jnp.cumsum(mask.astype(jnp.int32), axis=1)
    q = jnp.arange(1, SHORTLIST_K + 1, dtype=jnp.int32)
    return jax.vmap(lambda c: jnp.searchsorted(c, q, side="left"))(cs)


def _matmul_nt_kernel(a_ref, b_ref, o_ref):
    o_ref[...] = lax.dot_general(
        a_ref[...], b_ref[...],
        dimension_numbers=(((1,), (1,)), ((), ())),
        preferred_element_type=jnp.float32,
    )


def _matmul_nt(a, b_table, n_cols, tile):
    m = a.shape[0]
    return pl.pallas_call(
        _matmul_nt_kernel,
        grid=(pl.cdiv(n_cols, tile),),
        in_specs=[
            pl.BlockSpec((m, D), lambda i: (0, 0)),
            pl.BlockSpec((tile, D), lambda i: (i, 0)),
        ],
        out_specs=pl.BlockSpec((m, tile), lambda i: (0, i)),
        out_shape=jax.ShapeDtypeStruct((m, n_cols), jnp.float32),
    )(a, b_table)


def kernel(tokens, embed_table, memory, field_w, scout_w):
    E = jnp.take(embed_table, tokens.reshape(-1), axis=0)

    scores, bmax = _scores_and_blockmax(E, memory)
    tblk = _kth_largest_key(bmax, SHORTLIST_K, axis=0)

    sel = _compact64((_skey(bmax) >= tblk).T)
    scores3 = scores.reshape(NTOK, NB, BLK)
    cand_s = jnp.take_along_axis(scores3, sel[:, :, None], axis=1)
    cand_flat = cand_s.reshape(NTOK, SHORTLIST_K * BLK)

    v64 = _kth_largest_key(cand_flat, SHORTLIST_K, axis=1)
    pos = _compact64(_skey(cand_flat) >= v64)
    Kset = jnp.take_along_axis(sel, pos // BLK, axis=1) * BLK + pos % BLK
    cand = jnp.take(memory, Kset, axis=0)

    Eb = E.reshape(B, S, D)
    candb = cand.reshape(B, S, SHORTLIST_K, D)
    proj = jnp.einsum('bsd,de->bse', Eb, scout_w)
    sims = jnp.einsum('bse,bske->bsk', proj, candb)
    P = jax.nn.softmax(sims, axis=-1)
    Y = jnp.einsum('bsk,bske->bse', P, candb)
    Lam = jnp.zeros_like(Y)
    W = field_w / jnp.maximum(1.0, jnp.linalg.norm(field_w))
    X = Eb

    def field_apply(Yv):
        out = jnp.zeros_like(Yv)
        for i, off in enumerate(range(-BAND, BAND + 1)):
            out = out + jnp.roll(Yv, shift=off, axis=1) * W[i]
        return out

    prev_energy = jnp.array(jnp.inf, dtype=jnp.float32)
    done = jnp.array(False)
    for t in range(T_TRAIN):
        frac = t / max(T_TRAIN - 1, 1)
        beta = BETA_START + frac * (BETA_END - BETA_START)
        tau = TAU_START + frac * (TAU_END - TAU_START)
        FY = field_apply(Y)
        Lam_n = Lam + tau * (FY - X)
        resid = Y - X + Lam_n
        g = jnp.einsum('bse,bske->bsk', resid, candb)
        P_n = jax.nn.softmax(jnp.log(P + 1e-9) - beta * g, axis=-1)
        Y_n = jnp.einsum('bsk,bske->bse', P_n, candb)
        energy = 0.5 * jnp.mean(jnp.sum((Y_n - X) ** 2, axis=-1)) \
               + 0.5 * jnp.mean(jnp.sum((field_apply(Y_n) - X) ** 2, axis=-1))
        e = energy.astype(jnp.float32)
        Lam = jnp.where(done, Lam, Lam_n)
        P = jnp.where(done, P, P_n)
        Y = jnp.where(done, Y, Y_n)
        if t > 0:
            rel = jnp.abs(prev_energy - e) / jnp.maximum(jnp.abs(prev_energy), 1e-6)
            done = done | (rel <= EARLY_EXIT_TOL)
        prev_energy = jnp.where(done, prev_energy, e)

    Yf = Y.reshape(NTOK, D)
    logits = _matmul_nt(Yf, memory, VOCAB, LOGIT_TILE)
    return logits.reshape(B, S, VOCAB)


# speedup vs baseline: 26.4866x; 1.1821x over previous
import functools

import jax
import jax.numpy as jnp
from jax import lax
from jax.experimental import pallas as pl
from jax.experimental.pallas import tpu as pltpu

VOCAB = 100000
D = 64
K_MEM = 120000
SHORTLIST_K = 64
T_TRAIN = 4
BAND = 4
BETA_START, BETA_END = 1.0, 5.0
TAU_START, TAU_END = 1.0, 0.1
EARLY_EXIT_TOL = 1e-4
B, S = 32, 8
NTOK = B * S

SCORE_TILE = 512
BLK = 64
N_TILES = 235
K_PAD = N_TILES * SCORE_TILE
NB = K_PAD // BLK
NB_OUT = 1888
NEG_INF = float("-inf")

LOGIT_TILE = 512


def _skey(x):
    b = lax.bitcast_convert_type(x, jnp.int32)
    return jnp.where(b >= 0, b, b ^ jnp.int32(0x7FFFFFFF))


def _score_bmax_kernel(e_ref, mem_ref, s_ref, bm_ref):
    i = pl.program_id(0)
    s = lax.dot_general(
        e_ref[...], mem_ref[...],
        dimension_numbers=(((1,), (1,)), ((), ())),
        preferred_element_type=jnp.float32,
    )
    col = i * SCORE_TILE + lax.broadcasted_iota(jnp.int32, s.shape, 1)
    s = jnp.where(col < K_MEM, s, NEG_INF)
    s_ref[...] = s
    nblk = SCORE_TILE // BLK
    parts = [jnp.max(s[:, j * BLK:(j + 1) * BLK], axis=1, keepdims=True)
             for j in range(nblk)]
    bm = jnp.concatenate(parts, axis=1)
    off = pl.multiple_of(i * nblk, 8)
    bm_ref[pl.ds(off, nblk), :] = bm.T

    @pl.when(i == 0)
    def _fill_tail():
        bm_ref[pl.ds(NB, NB_OUT - NB), :] = jnp.full(
            (NB_OUT - NB, NTOK), NEG_INF, jnp.float32)


def _scores_and_blockmax(E, memory):
    return pl.pallas_call(
        _score_bmax_kernel,
        grid=(N_TILES,),
        in_specs=[
            pl.BlockSpec((NTOK, D), lambda i: (0, 0)),
            pl.BlockSpec((SCORE_TILE, D), lambda i: (i, 0)),
        ],
        out_specs=[
            pl.BlockSpec((NTOK, SCORE_TILE), lambda i: (0, i)),
            pl.BlockSpec((NB_OUT, NTOK), lambda i: (0, 0)),
        ],
        out_shape=[
            jax.ShapeDtypeStruct((NTOK, K_PAD), jnp.float32),
            jax.ShapeDtypeStruct((NB_OUT, NTOK), jnp.float32),
        ],
    )(E, memory)


def _bisect_kernel(x_ref, o_ref, *, kwant, axis):
    keys = _skey(x_ref[...])
    oshape = o_ref.shape

    def body(_, lohi):
        lo, hi = lohi
        mid = (lo >> 1) + (hi >> 1) + (lo & hi & 1)
        cnt = jnp.sum((keys >= mid).astype(jnp.int32), axis=axis, keepdims=True)
        ge = cnt >= kwant
        return jnp.where(ge, mid, lo), jnp.where(ge, hi, mid)

    lo0 = jnp.full(oshape, jnp.int32(-2147483648))
    hi0 = jnp.full(oshape, jnp.int32(2147483647))
    lo, _ = lax.fori_loop(0, 32, body, (lo0, hi0))
    o_ref[...] = lo


def _kth_largest_key(x, kwant, axis):
    m, n = x.shape
    oshape = (m, 1) if axis == 1 else (1, n)
    return pl.pallas_call(
        functools.partial(_bisect_kernel, kwant=kwant, axis=axis),
        out_shape=jax.ShapeDtypeStruct(oshape, jnp.int32),
    )(x)


def _compact64(mask):
    cs = jnp.cumsum(mask.astype(jnp.int32), axis=1)
    q = jnp.arange(1, SHORTLIST_K + 1, dtype=jnp.int32)
    return jax.vmap(lambda c: jnp.searchsorted(c, q, side="left"))(cs)


def _pdhg_kernel(e_ref, cand_ref, fw_ref, sw_ref, y_ref):
    X = e_ref[...].reshape(B, S, D)
    cand = cand_ref[...].reshape(B, S, SHORTLIST_K, D)
    W = fw_ref[...] / jnp.maximum(1.0, jnp.sqrt(jnp.sum(fw_ref[...] ** 2)))

    proj = lax.dot_general(
        e_ref[...], sw_ref[...],
        dimension_numbers=(((1,), (0,)), ((), ())),
        preferred_element_type=jnp.float32,
    ).reshape(B, S, 1, D)

    def dotk(a, c):
        return jnp.sum(a * c, axis=-1)

    def softmax(x):
        m = jnp.max(x, axis=-1, keepdims=True)
        ex = jnp.exp(x - m)
        return ex / jnp.sum(ex, axis=-1, keepdims=True)

    def field_apply(Yv):
        out = jnp.zeros_like(Yv)
        for i, off in enumerate(range(-BAND, BAND + 1)):
            if off == 0:
                rolled = Yv
            elif off > 0:
                rolled = jnp.concatenate([Yv[:, S - off:, :], Yv[:, :S - off, :]], axis=1)
            else:
                rolled = jnp.concatenate([Yv[:, -off:, :], Yv[:, :-off, :]], axis=1)
            out = out + rolled * W[i]
        return out

    sims = dotk(proj, cand)
    P = softmax(sims)
    Y = jnp.sum(P[..., None] * cand, axis=2)
    Lam = jnp.zeros_like(Y)

    prev_energy = jnp.float32(jnp.inf)
    done = jnp.array(False)
    for t in range(T_TRAIN):
        frac = t / max(T_TRAIN - 1, 1)
        beta = BETA_START + frac * (BETA_END - BETA_START)
        tau = TAU_START + frac * (TAU_END - TAU_START)
        FY = field_apply(Y)
        Lam_n = Lam + tau * (FY - X)
        resid = Y - X + Lam_n
        g = dotk(resid[:, :, None, :], cand)
        P_n = softmax(jnp.log(P + 1e-9) - beta * g)
        Y_n = jnp.sum(P_n[..., None] * cand, axis=2)
        energy = 0.5 * jnp.mean(jnp.sum((Y_n - X) ** 2, axis=-1)) \
               + 0.5 * jnp.mean(jnp.sum((field_apply(Y_n) - X) ** 2, axis=-1))
        e = energy.astype(jnp.float32)
        Lam = jnp.where(done, Lam, Lam_n)
        P = jnp.where(done, P, P_n)
        Y = jnp.where(done, Y, Y_n)
        if t > 0:
            rel = jnp.abs(prev_energy - e) / jnp.maximum(jnp.abs(prev_energy), 1e-6)
            done = done | (rel <= EARLY_EXIT_TOL)
        prev_energy = jnp.where(done, prev_energy, e)

    y_ref[...] = Y.reshape(NTOK, D)


def _pdhg(E, cand, field_w, scout_w):
    return pl.pallas_call(
        _pdhg_kernel,
        out_shape=jax.ShapeDtypeStruct((NTOK, D), jnp.float32),
    )(E, cand, field_w, scout_w)


def _matmul_nt_kernel(a_ref, b_ref, o_ref):
    o_ref[...] = lax.dot_general(
        a_ref[...], b_ref[...],
        dimension_numbers=(((1,), (1,)), ((), ())),
        preferred_element_type=jnp.float32,
    )


def _matmul_nt(a, b_table, n_cols, tile):
    m = a.shape[0]
    return pl.pallas_call(
        _matmul_nt_kernel,
        grid=(pl.cdiv(n_cols, tile),),
        in_specs=[
            pl.BlockSpec((m, D), lambda i: (0, 0)),
            pl.BlockSpec((tile, D), lambda i: (i, 0)),
        ],
        out_specs=pl.BlockSpec((m, tile), lambda i: (0, i)),
        out_shape=jax.ShapeDtypeStruct((m, n_cols), jnp.float32),
    )(a, b_table)


def kernel(tokens, embed_table, memory, field_w, scout_w):
    E = jnp.take(embed_table, tokens.reshape(-1), axis=0)

    scores, bmax = _scores_and_blockmax(E, memory)
    tblk = _kth_largest_key(bmax, SHORTLIST_K, axis=0)

    sel = _compact64((_skey(bmax) >= tblk).T)
    scores3 = scores.reshape(NTOK, NB, BLK)
    cand_s = jnp.take_along_axis(scores3, sel[:, :, None], axis=1)
    cand_flat = cand_s.reshape(NTOK, SHORTLIST_K * BLK)

    v64 = _kth_largest_key(cand_flat, SHORTLIST_K, axis=1)
    pos = _compact64(_skey(cand_flat) >= v64)
    Kset = jnp.take_along_axis(sel, pos // BLK, axis=1) * BLK + pos % BLK
    cand = jnp.take(memory, Kset, axis=0)

    Yf = _pdhg(E, cand, field_w, scout_w)
    logits = _matmul_nt(Yf, memory, VOCAB, LOGIT_TILE)
    return logits.reshape(B, S, VOCAB)


# speedup vs baseline: 32.7471x; 1.2364x over previous
import functools

import jax
import jax.numpy as jnp
from jax import lax
from jax.experimental import pallas as pl
from jax.experimental.pallas import tpu as pltpu
from jax.experimental.pallas import tpu_sc as plsc

NCORES = 2
NSUB = 16
NW = NCORES * NSUB
TOK_PER_W = 8

VOCAB = 100000
D = 64
K_MEM = 120000
SHORTLIST_K = 64
T_TRAIN = 4
BAND = 4
BETA_START, BETA_END = 1.0, 5.0
TAU_START, TAU_END = 1.0, 0.1
EARLY_EXIT_TOL = 1e-4
B, S = 32, 8
NTOK = B * S

SCORE_TILE = 512
BLK = 64
N_TILES = 235
K_PAD = N_TILES * SCORE_TILE
NB = K_PAD // BLK
NB_OUT = 1920
NEG_INF = float("-inf")

LOGIT_TILE = 512


def _skey(x):
    b = lax.bitcast_convert_type(x, jnp.int32)
    return jnp.where(b >= 0, b, b ^ jnp.int32(0x7FFFFFFF))


def _score_bmax_kernel(e_ref, mem_ref, s_ref, bm_ref):
    i = pl.program_id(0)
    s = lax.dot_general(
        e_ref[...], mem_ref[...],
        dimension_numbers=(((1,), (1,)), ((), ())),
        preferred_element_type=jnp.float32,
    )
    col = i * SCORE_TILE + lax.broadcasted_iota(jnp.int32, s.shape, 1)
    s = jnp.where(col < K_MEM, s, NEG_INF)
    s_ref[...] = s
    nblk = SCORE_TILE // BLK
    parts = [jnp.max(s[:, j * BLK:(j + 1) * BLK], axis=1, keepdims=True)
             for j in range(nblk)]
    bm = jnp.concatenate(parts, axis=1)
    off = pl.multiple_of(i * nblk, 8)
    bm_ref[pl.ds(off, nblk), :] = bm.T

    @pl.when(i == 0)
    def _fill_tail():
        bm_ref[pl.ds(NB, NB_OUT - NB), :] = jnp.full(
            (NB_OUT - NB, NTOK), NEG_INF, jnp.float32)


def _scores_and_blockmax(E, memory):
    return pl.pallas_call(
        _score_bmax_kernel,
        grid=(N_TILES,),
        in_specs=[
            pl.BlockSpec((NTOK, D), lambda i: (0, 0)),
            pl.BlockSpec((SCORE_TILE, D), lambda i: (i, 0)),
        ],
        out_specs=[
            pl.BlockSpec((NTOK, SCORE_TILE), lambda i: (0, i)),
            pl.BlockSpec((NB_OUT, NTOK), lambda i: (0, 0)),
        ],
        out_shape=[
            jax.ShapeDtypeStruct((NTOK, K_PAD), jnp.float32),
            jax.ShapeDtypeStruct((NB_OUT, NTOK), jnp.float32),
        ],
    )(E, memory)


def _bisect_kernel(x_ref, o_ref, *, kwant, axis):
    keys = _skey(x_ref[...])
    oshape = o_ref.shape

    def body(_, lohi):
        lo, hi = lohi
        mid = (lo >> 1) + (hi >> 1) + (lo & hi & 1)
        cnt = jnp.sum((keys >= mid).astype(jnp.int32), axis=axis, keepdims=True)
        ge = cnt >= kwant
        return jnp.where(ge, mid, lo), jnp.where(ge, hi, mid)

    lo0 = jnp.full(oshape, jnp.int32(-2147483648))
    hi0 = jnp.full(oshape, jnp.int32(2147483647))
    lo, _ = lax.fori_loop(0, 32, body, (lo0, hi0))
    o_ref[...] = lo


def _kth_largest_key(x, kwant, axis):
    m, n = x.shape
    oshape = (m, 1) if axis == 1 else (1, n)
    return pl.pallas_call(
        functools.partial(_bisect_kernel, kwant=kwant, axis=axis),
        out_shape=jax.ShapeDtypeStruct(oshape, jnp.int32),
    )(x)


def _bisect_t_kernel(x_ref, o_ref, xt_ref, *, kwant):
    x = x_ref[...]
    keys = _skey(x)

    def body(_, lohi):
        lo, hi = lohi
        mid = (lo >> 1) + (hi >> 1) + (lo & hi & 1)
        cnt = jnp.sum((keys >= mid).astype(jnp.int32), axis=0, keepdims=True)
        ge = cnt >= kwant
        return jnp.where(ge, mid, lo), jnp.where(ge, hi, mid)

    lo0 = jnp.full((1, x.shape[1]), jnp.int32(-2147483648))
    hi0 = jnp.full((1, x.shape[1]), jnp.int32(2147483647))
    lo, _ = lax.fori_loop(0, 32, body, (lo0, hi0))
    o_ref[...] = lo
    xt_ref[...] = x.T


def _kth_largest_key_t(x, kwant):
    n, m = x.shape
    return pl.pallas_call(
        functools.partial(_bisect_t_kernel, kwant=kwant),
        out_shape=[
            jax.ShapeDtypeStruct((1, m), jnp.int32),
            jax.ShapeDtypeStruct((m, n), jnp.float32),
        ],
    )(x)


def _sc_mesh():
    return plsc.VectorSubcoreMesh(core_axis_name="c", subcore_axis_name="s")


def _sc_wid():
    return lax.axis_index("s") * NCORES + lax.axis_index("c")


def _skey16(v):
    b = lax.bitcast_convert_type(v, jnp.int32)
    return jnp.where(b >= 0, b, b ^ jnp.int32(0x7FFFFFFF))


def _splat(ref, i):
    v = ref[pl.ds(i, 16)]
    w = jnp.where(lax.iota(jnp.int32, 16) == 0, v, jnp.int32(-2147483648))
    return plsc.cummax(w)


def _compact_scatter(keys, thr, ids, outbuf, base):
    m = keys >= thr
    incs = plsc.cumsum(m.astype(jnp.int32))
    tgt = base + incs - 1
    m2 = jnp.logical_and(m, tgt < SHORTLIST_K)
    plsc.store_scatter(outbuf, [jnp.maximum(tgt, 0)], ids, mask=m2)
    return base + plsc.all_reduce_population_count(m)


def _sc_compact_blocks(bmax_t, tblk):
    n_chunks = NB_OUT // 16

    def body(bm_hbm, tb_hbm, sel_hbm, bm_v, tb_v, sel_v):
        wid = _sc_wid()
        pltpu.sync_copy(tb_hbm, tb_v.at[pl.ds(0, NTOK)])

        def per_token(j, _):
            t = wid * TOK_PER_W + j
            pltpu.sync_copy(bm_hbm.at[t], bm_v)
            thr = _splat(tb_v, t)

            def chunk(c, base):
                v = bm_v[pl.ds(c * 16, 16)]
                ids = c * 16 + lax.iota(jnp.int32, 16)
                return _compact_scatter(_skey16(v), thr, ids, sel_v, base)

            lax.fori_loop(0, n_chunks, chunk, jnp.zeros((16,), jnp.int32))
            pltpu.sync_copy(sel_v, sel_hbm.at[t])
            return 0

        lax.fori_loop(0, TOK_PER_W, per_token, 0)

    return pl.kernel(
        body,
        out_type=jax.ShapeDtypeStruct((NTOK, 128), jnp.int32),
        mesh=_sc_mesh(),
        compiler_params=pltpu.CompilerParams(needs_layout_passes=False),
        scratch_types=[
            pltpu.VMEM((NB_OUT,), jnp.float32),
            pltpu.VMEM((NTOK + 16,), jnp.int32),
            pltpu.VMEM((128,), jnp.int32),
        ],
    )(bmax_t, tblk)


def _sc_compact_final(cand_s, sel, v64):
    ncand = SHORTLIST_K * BLK
    n_chunks = ncand // 16

    def body(cs_hbm, sel_hbm, v64_hbm, kset_hbm, cs_v, sel_v, vv, kbuf):
        wid = _sc_wid()
        pltpu.sync_copy(v64_hbm, vv.at[pl.ds(0, NTOK)])

        def per_token(j, _):
            t = wid * TOK_PER_W + j
            pltpu.sync_copy(cs_hbm.at[t], cs_v)
            pltpu.sync_copy(sel_hbm.at[t], sel_v)
            thr = _splat(vv, t)

            def chunk(c, base):
                v = cs_v[pl.ds(c * 16, 16)]
                blk = _splat(sel_v, c // (BLK // 16))
                gidx = blk * BLK + (c % (BLK // 16)) * 16 + lax.iota(jnp.int32, 16)
                return _compact_scatter(_skey16(v), thr, gidx, kbuf, base)

            lax.fori_loop(0, n_chunks, chunk, jnp.zeros((16,), jnp.int32))
            pltpu.sync_copy(kbuf, kset_hbm.at[t])
            return 0

        lax.fori_loop(0, TOK_PER_W, per_token, 0)

    return pl.kernel(
        body,
        out_type=jax.ShapeDtypeStruct((NTOK, 128), jnp.int32),
        mesh=_sc_mesh(),
        compiler_params=pltpu.CompilerParams(needs_layout_passes=False),
        scratch_types=[
            pltpu.VMEM((ncand,), jnp.float32),
            pltpu.VMEM((128,), jnp.int32),
            pltpu.VMEM((NTOK + 16,), jnp.int32),
            pltpu.VMEM((128,), jnp.int32),
        ],
    )(cand_s, sel, v64)


def _pdhg_kernel(e_ref, cand_ref, fw_ref, sw_ref, y_ref):
    X = e_ref[...].reshape(B, S, D)
    cand = cand_ref[...].reshape(B, S, SHORTLIST_K, D)
    W = fw_ref[...] / jnp.maximum(1.0, jnp.sqrt(jnp.sum(fw_ref[...] ** 2)))

    proj = lax.dot_general(
        e_ref[...], sw_ref[...],
        dimension_numbers=(((1,), (0,)), ((), ())),
        preferred_element_type=jnp.float32,
    ).reshape(B, S, 1, D)

    def dotk(a, c):
        return jnp.sum(a * c, axis=-1)

    def softmax(x):
        m = jnp.max(x, axis=-1, keepdims=True)
        ex = jnp.exp(x - m)
        return ex / jnp.sum(ex, axis=-1, keepdims=True)

    def field_apply(Yv):
        out = jnp.zeros_like(Yv)
        for i, off in enumerate(range(-BAND, BAND + 1)):
            if off == 0:
                rolled = Yv
            elif off > 0:
                rolled = jnp.concatenate([Yv[:, S - off:, :], Yv[:, :S - off, :]], axis=1)
            else:
                rolled = jnp.concatenate([Yv[:, -off:, :], Yv[:, :-off, :]], axis=1)
            out = out + rolled * W[i]
        return out

    sims = dotk(proj, cand)
    P = softmax(sims)
    Y = jnp.sum(P[..., None] * cand, axis=2)
    Lam = jnp.zeros_like(Y)

    prev_energy = jnp.float32(jnp.inf)
    done = jnp.array(False)
    for t in range(T_TRAIN):
        frac = t / max(T_TRAIN - 1, 1)
        beta = BETA_START + frac * (BETA_END - BETA_START)
        tau = TAU_START + frac * (TAU_END - TAU_START)
        FY = field_apply(Y)
        Lam_n = Lam + tau * (FY - X)
        resid = Y - X + Lam_n
        g = dotk(resid[:, :, None, :], cand)
        P_n = softmax(jnp.log(P + 1e-9) - beta * g)
        Y_n = jnp.sum(P_n[..., None] * cand, axis=2)
        energy = 0.5 * jnp.mean(jnp.sum((Y_n - X) ** 2, axis=-1)) \
               + 0.5 * jnp.mean(jnp.sum((field_apply(Y_n) - X) ** 2, axis=-1))
        e = energy.astype(jnp.float32)
        Lam = jnp.where(done, Lam, Lam_n)
        P = jnp.where(done, P, P_n)
        Y = jnp.where(done, Y, Y_n)
        if t > 0:
            rel = jnp.abs(prev_energy - e) / jnp.maximum(jnp.abs(prev_energy), 1e-6)
            done = done | (rel <= EARLY_EXIT_TOL)
        prev_energy = jnp.where(done, prev_energy, e)

    y_ref[...] = Y.reshape(NTOK, D)


def _pdhg(E, cand, field_w, scout_w):
    return pl.pallas_call(
        _pdhg_kernel,
        out_shape=jax.ShapeDtypeStruct((NTOK, D), jnp.float32),
    )(E, cand, field_w, scout_w)


def _matmul_nt_kernel(a_ref, b_ref, o_ref):
    o_ref[...] = lax.dot_general(
        a_ref[...], b_ref[...],
        dimension_numbers=(((1,), (1,)), ((), ())),
        preferred_element_type=jnp.float32,
    )


def _matmul_nt(a, b_table, n_cols, tile):
    m = a.shape[0]
    return pl.pallas_call(
        _matmul_nt_kernel,
        grid=(pl.cdiv(n_cols, tile),),
        in_specs=[
            pl.BlockSpec((m, D), lambda i: (0, 0)),
            pl.BlockSpec((tile, D), lambda i: (i, 0)),
        ],
        out_specs=pl.BlockSpec((m, tile), lambda i: (0, i)),
        out_shape=jax.ShapeDtypeStruct((m, n_cols), jnp.float32),
    )(a, b_table)


def kernel(tokens, embed_table, memory, field_w, scout_w):
    E = jnp.take(embed_table, tokens.reshape(-1), axis=0)

    scores, bmax = _scores_and_blockmax(E, memory)
    tblk, bmax_t = _kth_largest_key_t(bmax, SHORTLIST_K)

    self_f = _sc_compact_blocks(bmax_t, tblk.reshape(NTOK))
    sel = self_f[:, :SHORTLIST_K]
    scores3 = scores.reshape(NTOK, NB, BLK)
    cand_s = jnp.take_along_axis(scores3, sel[:, :, None], axis=1)
    cand_flat = cand_s.reshape(NTOK, SHORTLIST_K * BLK)

    v64 = _kth_largest_key(cand_flat, SHORTLIST_K, axis=1)
    kset_f = _sc_compact_final(cand_flat, self_f, v64.reshape(NTOK))
    Kset = kset_f[:, :SHORTLIST_K]
    cand = jnp.take(memory, Kset, axis=0)

    Yf = _pdhg(E, cand, field_w, scout_w)
    logits = _matmul_nt(Yf, memory, VOCAB, LOGIT_TILE)
    return logits.reshape(B, S, VOCAB)


# speedup vs baseline: 33.3733x; 1.0191x over previous
import functools

import jax
import jax.numpy as jnp
from jax import lax
from jax.experimental import pallas as pl
from jax.experimental.pallas import tpu as pltpu
from jax.experimental.pallas import tpu_sc as plsc

NCORES = 2
NSUB = 16
NW = NCORES * NSUB
TOK_PER_W = 8

VOCAB = 100000
D = 64
K_MEM = 120000
SHORTLIST_K = 64
T_TRAIN = 4
BAND = 4
BETA_START, BETA_END = 1.0, 5.0
TAU_START, TAU_END = 1.0, 0.1
EARLY_EXIT_TOL = 1e-4
B, S = 32, 8
NTOK = B * S

SCORE_TILE = 512
BLK = 64
N_TILES = 235
K_PAD = N_TILES * SCORE_TILE
NB = K_PAD // BLK
NB_OUT = 1920
NEG_INF = float("-inf")

LOGIT_TILE = 512


def _skey(x):
    b = lax.bitcast_convert_type(x, jnp.int32)
    return jnp.where(b >= 0, b, b ^ jnp.int32(0x7FFFFFFF))


def _score_bmax_kernel(e_ref, mem_ref, s_ref, bm_ref):
    i = pl.program_id(0)
    s = lax.dot_general(
        e_ref[...], mem_ref[...],
        dimension_numbers=(((1,), (1,)), ((), ())),
        preferred_element_type=jnp.float32,
    )
    col = i * SCORE_TILE + lax.broadcasted_iota(jnp.int32, s.shape, 1)
    s = jnp.where(col < K_MEM, s, NEG_INF)
    s_ref[...] = s
    nblk = SCORE_TILE // BLK
    parts = [jnp.max(s[:, j * BLK:(j + 1) * BLK], axis=1, keepdims=True)
             for j in range(nblk)]
    bm = jnp.concatenate(parts, axis=1)
    off = pl.multiple_of(i * nblk, 8)
    bm_ref[pl.ds(off, nblk), :] = bm.T

    @pl.when(i == 0)
    def _fill_tail():
        bm_ref[pl.ds(NB, NB_OUT - NB), :] = jnp.full(
            (NB_OUT - NB, NTOK), NEG_INF, jnp.float32)


def _scores_and_blockmax(E, memory):
    return pl.pallas_call(
        _score_bmax_kernel,
        grid=(N_TILES,),
        in_specs=[
            pl.BlockSpec((NTOK, D), lambda i: (0, 0)),
            pl.BlockSpec((SCORE_TILE, D), lambda i: (i, 0)),
        ],
        out_specs=[
            pl.BlockSpec((NTOK, SCORE_TILE), lambda i: (0, i)),
            pl.BlockSpec((NB_OUT, NTOK), lambda i: (0, 0)),
        ],
        out_shape=[
            jax.ShapeDtypeStruct((NTOK, K_PAD), jnp.float32),
            jax.ShapeDtypeStruct((NB_OUT, NTOK), jnp.float32),
        ],
    )(E, memory)


def _bisect_kernel(x_ref, o_ref, *, kwant, axis):
    keys = _skey(x_ref[...])
    oshape = o_ref.shape

    def body(_, lohi):
        lo, hi = lohi
        mid = (lo >> 1) + (hi >> 1) + (lo & hi & 1)
        cnt = jnp.sum((keys >= mid).astype(jnp.int32), axis=axis, keepdims=True)
        ge = cnt >= kwant
        return jnp.where(ge, mid, lo), jnp.where(ge, hi, mid)

    lo0 = jnp.full(oshape, jnp.int32(-2147483648))
    hi0 = jnp.full(oshape, jnp.int32(2147483647))
    lo, _ = lax.fori_loop(0, 32, body, (lo0, hi0))
    o_ref[...] = lo


def _kth_largest_key(x, kwant, axis):
    m, n = x.shape
    oshape = (m, 1) if axis == 1 else (1, n)
    return pl.pallas_call(
        functools.partial(_bisect_kernel, kwant=kwant, axis=axis),
        out_shape=jax.ShapeDtypeStruct(oshape, jnp.int32),
    )(x)


def _bisect_t_kernel(x_ref, o_ref, xt_ref, *, kwant):
    x = x_ref[...]
    keys = _skey(x)

    def body(_, lohi):
        lo, hi = lohi
        mid = (lo >> 1) + (hi >> 1) + (lo & hi & 1)
        cnt = jnp.sum((keys >= mid).astype(jnp.int32), axis=0, keepdims=True)
        ge = cnt >= kwant
        return jnp.where(ge, mid, lo), jnp.where(ge, hi, mid)

    lo0 = jnp.full((1, x.shape[1]), jnp.int32(-2147483648))
    hi0 = jnp.full((1, x.shape[1]), jnp.int32(2147483647))
    lo, _ = lax.fori_loop(0, 32, body, (lo0, hi0))
    o_ref[...] = lo
    xt_ref[...] = x.T


def _kth_largest_key_t(x, kwant):
    n, m = x.shape
    return pl.pallas_call(
        functools.partial(_bisect_t_kernel, kwant=kwant),
        out_shape=[
            jax.ShapeDtypeStruct((1, m), jnp.int32),
            jax.ShapeDtypeStruct((m, n), jnp.float32),
        ],
    )(x)


def _sc_mesh():
    return plsc.VectorSubcoreMesh(core_axis_name="c", subcore_axis_name="s")


def _sc_wid():
    return lax.axis_index("s") * NCORES + lax.axis_index("c")


def _skey16(v):
    b = lax.bitcast_convert_type(v, jnp.int32)
    return jnp.where(b >= 0, b, b ^ jnp.int32(0x7FFFFFFF))


def _splat(ref, i):
    v = ref[pl.ds(i, 16)]
    w = jnp.where(lax.iota(jnp.int32, 16) == 0, v, jnp.int32(-2147483648))
    return plsc.cummax(w)


def _compact_scatter(keys, thr, ids, outbuf, base):
    m = keys >= thr
    incs = plsc.cumsum(m.astype(jnp.int32))
    tgt = base + incs - 1
    m2 = jnp.logical_and(m, tgt < SHORTLIST_K)
    plsc.store_scatter(outbuf, [jnp.maximum(tgt, 0)], ids, mask=m2)
    return base + plsc.all_reduce_population_count(m)


def _sc_compact_blocks(bmax_t, tblk):
    n_chunks = NB_OUT // 16

    def body(bm_hbm, tb_hbm, sel_hbm, bm_v, tb_v, sel_v):
        wid = _sc_wid()
        pltpu.sync_copy(tb_hbm, tb_v.at[pl.ds(0, NTOK)])

        def per_token(j, _):
            t = wid * TOK_PER_W + j
            pltpu.sync_copy(bm_hbm.at[t], bm_v)
            thr = _splat(tb_v, t)

            def chunk(c, base):
                v = bm_v[pl.ds(c * 16, 16)]
                ids = c * 16 + lax.iota(jnp.int32, 16)
                return _compact_scatter(_skey16(v), thr, ids, sel_v, base)

            lax.fori_loop(0, n_chunks, chunk, jnp.zeros((16,), jnp.int32))
            pltpu.sync_copy(sel_v, sel_hbm.at[t])
            return 0

        lax.fori_loop(0, TOK_PER_W, per_token, 0)

    return pl.kernel(
        body,
        out_type=jax.ShapeDtypeStruct((NTOK, 128), jnp.int32),
        mesh=_sc_mesh(),
        compiler_params=pltpu.CompilerParams(needs_layout_passes=False),
        scratch_types=[
            pltpu.VMEM((NB_OUT,), jnp.float32),
            pltpu.VMEM((NTOK + 16,), jnp.int32),
            pltpu.VMEM((128,), jnp.int32),
        ],
    )(bmax_t, tblk)


def _sc_compact_final(cand_s, sel, v64):
    ncand = SHORTLIST_K * BLK
    n_chunks = ncand // 16

    def body(cs_hbm, sel_hbm, v64_hbm, kset_hbm, cs_v, sel_v, vv, kbuf):
        wid = _sc_wid()
        pltpu.sync_copy(v64_hbm, vv.at[pl.ds(0, NTOK)])

        def per_token(j, _):
            t = wid * TOK_PER_W + j
            pltpu.sync_copy(cs_hbm.at[t], cs_v)
            pltpu.sync_copy(sel_hbm.at[t], sel_v)
            thr = _splat(vv, t)

            def chunk(c, base):
                v = cs_v[pl.ds(c * 16, 16)]
                blk = _splat(sel_v, c // (BLK // 16))
                gidx = blk * BLK + (c % (BLK // 16)) * 16 + lax.iota(jnp.int32, 16)
                return _compact_scatter(_skey16(v), thr, gidx, kbuf, base)

            lax.fori_loop(0, n_chunks, chunk, jnp.zeros((16,), jnp.int32))
            pltpu.sync_copy(kbuf, kset_hbm.at[t])
            return 0

        lax.fori_loop(0, TOK_PER_W, per_token, 0)

    return pl.kernel(
        body,
        out_type=jax.ShapeDtypeStruct((NTOK, 128), jnp.int32),
        mesh=_sc_mesh(),
        compiler_params=pltpu.CompilerParams(needs_layout_passes=False),
        scratch_types=[
            pltpu.VMEM((ncand,), jnp.float32),
            pltpu.VMEM((128,), jnp.int32),
            pltpu.VMEM((NTOK + 16,), jnp.int32),
            pltpu.VMEM((128,), jnp.int32),
        ],
    )(cand_s, sel, v64)


def _pdhg_kernel(e_ref, cand_ref, fw_ref, sw_ref, y_ref):
    X = e_ref[...].reshape(B, S, D)
    cand = cand_ref[...].reshape(B, S, SHORTLIST_K, D)
    W = fw_ref[...] / jnp.maximum(1.0, jnp.sqrt(jnp.sum(fw_ref[...] ** 2)))

    proj = lax.dot_general(
        e_ref[...], sw_ref[...],
        dimension_numbers=(((1,), (0,)), ((), ())),
        preferred_element_type=jnp.float32,
    ).reshape(B, S, 1, D)

    def dotk(a, c):
        return jnp.sum(a * c, axis=-1)

    def softmax(x):
        m = jnp.max(x, axis=-1, keepdims=True)
        ex = jnp.exp(x - m)
        return ex / jnp.sum(ex, axis=-1, keepdims=True)

    def field_apply(Yv):
        out = jnp.zeros_like(Yv)
        for i, off in enumerate(range(-BAND, BAND + 1)):
            if off == 0:
                rolled = Yv
            elif off > 0:
                rolled = jnp.concatenate([Yv[:, S - off:, :], Yv[:, :S - off, :]], axis=1)
            else:
                rolled = jnp.concatenate([Yv[:, -off:, :], Yv[:, :-off, :]], axis=1)
            out = out + rolled * W[i]
        return out

    sims = dotk(proj, cand)
    P = softmax(sims)
    Y = jnp.sum(P[..., None] * cand, axis=2)
    Lam = jnp.zeros_like(Y)

    prev_energy = jnp.float32(jnp.inf)
    done = jnp.array(False)
    for t in range(T_TRAIN):
        frac = t / max(T_TRAIN - 1, 1)
        beta = BETA_START + frac * (BETA_END - BETA_START)
        tau = TAU_START + frac * (TAU_END - TAU_START)
        FY = field_apply(Y)
        Lam_n = Lam + tau * (FY - X)
        resid = Y - X + Lam_n
        g = dotk(resid[:, :, None, :], cand)
        P_n = softmax(jnp.log(P + 1e-9) - beta * g)
        Y_n = jnp.sum(P_n[..., None] * cand, axis=2)
        energy = 0.5 * jnp.mean(jnp.sum((Y_n - X) ** 2, axis=-1)) \
               + 0.5 * jnp.mean(jnp.sum((field_apply(Y_n) - X) ** 2, axis=-1))
        e = energy.astype(jnp.float32)
        Lam = jnp.where(done, Lam, Lam_n)
        P = jnp.where(done, P, P_n)
        Y = jnp.where(done, Y, Y_n)
        if t > 0:
            rel = jnp.abs(prev_energy - e) / jnp.maximum(jnp.abs(prev_energy), 1e-6)
            done = done | (rel <= EARLY_EXIT_TOL)
        prev_energy = jnp.where(done, prev_energy, e)

    y_ref[...] = Y.reshape(NTOK, D)


def _pdhg(E, cand, field_w, scout_w):
    return pl.pallas_call(
        _pdhg_kernel,
        out_shape=jax.ShapeDtypeStruct((NTOK, D), jnp.float32),
    )(E, cand, field_w, scout_w)


def _matmul_nt_kernel(a_ref, b_ref, o_ref):
    o_ref[...] = lax.dot_general(
        a_ref[...], b_ref[...],
        dimension_numbers=(((1,), (1,)), ((), ())),
        preferred_element_type=jnp.float32,
    )


def _matmul_nt(a, b_table, n_cols, tile):
    m = a.shape[0]
    return pl.pallas_call(
        _matmul_nt_kernel,
        grid=(pl.cdiv(n_cols, tile),),
        in_specs=[
            pl.BlockSpec((m, D), lambda i: (0, 0)),
            pl.BlockSpec((tile, D), lambda i: (i, 0)),
        ],
        out_specs=pl.BlockSpec((m, tile), lambda i: (0, i)),
        out_shape=jax.ShapeDtypeStruct((m, n_cols), jnp.float32),
    )(a, b_table)


def kernel(tokens, embed_table, memory, field_w, scout_w):
    E = jnp.take(embed_table, tokens.reshape(-1), axis=0, mode="clip")

    scores, bmax = _scores_and_blockmax(E, memory)
    tblk, bmax_t = _kth_largest_key_t(bmax, SHORTLIST_K)

    self_f = _sc_compact_blocks(bmax_t, tblk.reshape(NTOK))
    sel = self_f[:, :SHORTLIST_K]
    scores3 = scores.reshape(NTOK, NB, BLK)
    cand_s = jnp.take_along_axis(scores3, sel[:, :, None], axis=1)
    cand_flat = cand_s.reshape(NTOK, SHORTLIST_K * BLK)

    v64 = _kth_largest_key(cand_flat, SHORTLIST_K, axis=1)
    kset_f = _sc_compact_final(cand_flat, self_f, v64.reshape(NTOK))
    Kset = kset_f[:, :SHORTLIST_K]
    cand = jnp.take(memory, Kset, axis=0, mode="clip")

    Yf = _pdhg(E, cand, field_w, scout_w)
    logits = _matmul_nt(Yf, memory, VOCAB, LOGIT_TILE)
    return logits.reshape(B, S, VOCAB)
